# Initial kernel scaffold; baseline (speedup 1.0000x reference)
import jax
import jax.numpy as jnp
from jax.experimental import pallas as pl


def kernel(label_lists, table):
    raise NotImplementedError("write your pallas kernel here")


# speedup vs baseline: 1.6771x; 1.6771x over previous
import functools

import jax
import jax.numpy as jnp
from jax import lax
from jax.experimental import pallas as pl
from jax.experimental.pallas import tpu as pltpu
from jax.experimental.pallas import tpu_sc as plsc

B = 16384
H = 20
D = 32
NC = 2
NS = 16
NW = NC * NS
ROWS_W = B // NW
CHUNK = 128
NCHUNK = ROWS_W // CHUNK
IPG = 128
GATHERS = CHUNK * H // IPG
IDX_ROWS_W = ROWS_W * H // IPG


def _embed_body(idx_hbm, table_hbm, out_hbm, idx_v, buf, outbuf, sem):
    wid = lax.axis_index("s") * NC + lax.axis_index("c")
    base = wid * ROWS_W

    pltpu.sync_copy(idx_hbm.at[pl.ds(wid * IDX_ROWS_W, IDX_ROWS_W)], idx_v)

    def chunk_body(c, carry):
        copies = []
        for k in range(GATHERS):
            copies.append(pltpu.async_copy(
                table_hbm.at[idx_v.at[c * GATHERS + k]],
                buf.at[pl.ds(k * IPG, IPG)],
                sem))
        for cp in copies:
            cp.wait()

        def red_body(r, carry2):
            for half in range(2):
                sl = pl.ds(half * 16, 16)
                v = buf[r * H, sl]
                for j in range(1, H):
                    v = v + buf[r * H + j, sl]
                outbuf[r, sl] = v * (1.0 / H)
            return carry2

        lax.fori_loop(0, CHUNK, red_body, 0)
        pltpu.sync_copy(outbuf, out_hbm.at[pl.ds(base + c * CHUNK, CHUNK)])
        return carry

    lax.fori_loop(0, NCHUNK, chunk_body, 0)


_embed = functools.partial(
    pl.kernel,
    out_type=jax.ShapeDtypeStruct((B, D), jnp.float32),
    mesh=plsc.VectorSubcoreMesh(core_axis_name="c", subcore_axis_name="s"),
    compiler_params=pltpu.CompilerParams(use_tc_tiling_on_sc=False),
    scratch_types=[
        pltpu.VMEM((B * H // IPG // NW, IPG), jnp.int32),
        pltpu.VMEM((CHUNK * H, D), jnp.float32),
        pltpu.VMEM((CHUNK, D), jnp.float32),
        pltpu.SemaphoreType.DMA,
    ],
)(_embed_body)


def kernel(label_lists, table):
    idx = label_lists.astype(jnp.int32).reshape(B * H // IPG, IPG)
    return _embed(idx, table)


# speedup vs baseline: 1.7120x; 1.0208x over previous
import functools

import jax
import jax.numpy as jnp
from jax import lax
from jax.experimental import pallas as pl
from jax.experimental.pallas import tpu as pltpu
from jax.experimental.pallas import tpu_sc as plsc

NUM_ROWS = 1000000
B = 16384
H = 20
D = 32
NC = 2
NS = 16
NW = NC * NS
ROWS_W = B // NW
CHUNK = 128
NCHUNK = ROWS_W // CHUNK
IPG = 128
GATHERS = CHUNK * H // IPG
IDX_ROWS_W = ROWS_W * H // IPG


def _embed_body(idx_hbm, table_hbm, out_hbm, idx_v, buf, outbuf, sem):
    wid = lax.axis_index("s") * NC + lax.axis_index("c")
    base = wid * ROWS_W

    pltpu.sync_copy(idx_hbm.at[pl.ds(wid * IDX_ROWS_W, IDX_ROWS_W)], idx_v)

    def scale_body(row, carry):
        for g in range(8):
            sl = pl.ds(g * 16, 16)
            idx_v[row, sl] = idx_v[row, sl] * 4
        return carry

    lax.fori_loop(0, IDX_ROWS_W, scale_body, 0)

    def chunk_body(c, carry):
        copies = []
        for k in range(GATHERS):
            copies.append(pltpu.async_copy(
                table_hbm.at[idx_v.at[c * GATHERS + k]],
                buf.at[pl.ds(k * IPG, IPG)],
                sem))
        for cp in copies:
            cp.wait()

        def red_body(r, carry2):
            for half in range(2):
                sl = pl.ds(half * 16, 16)
                v = buf[r * H, sl]
                for j in range(1, H):
                    v = v + buf[r * H + j, sl]
                outbuf[r, sl] = v * (1.0 / H)
            return carry2

        lax.fori_loop(0, CHUNK, red_body, 0)
        pltpu.sync_copy(outbuf, out_hbm.at[pl.ds(base + c * CHUNK, CHUNK)])
        return carry

    lax.fori_loop(0, NCHUNK, chunk_body, 0)


_embed = functools.partial(
    pl.kernel,
    out_type=jax.ShapeDtypeStruct((B, D), jnp.float32),
    mesh=plsc.VectorSubcoreMesh(core_axis_name="c", subcore_axis_name="s"),
    compiler_params=pltpu.CompilerParams(use_tc_tiling_on_sc=False),
    scratch_types=[
        pltpu.VMEM((B * H // IPG // NW, IPG), jnp.int32),
        pltpu.VMEM((CHUNK * H, D), jnp.float32),
        pltpu.VMEM((CHUNK, D), jnp.float32),
        pltpu.SemaphoreType.DMA,
    ],
)(_embed_body)


def kernel(label_lists, table):
    idx = label_lists.astype(jnp.int32).reshape(B * H // IPG, IPG)
    tpad = jnp.pad(table, ((0, 0), (0, 96))).reshape(4 * NUM_ROWS, D)
    return _embed(idx, tpad)
